# baseline (speedup 1.0000x reference)
import functools

import jax
import jax.numpy as jnp
from jax import lax
from jax.experimental import pallas as pl
from jax.experimental.pallas import tpu as pltpu
from jax.experimental.pallas import tpu_sc as plsc

VOCAB = 1000000
EMBED_DIM = 64
BATCH = 16384
HIST = 200

_INFO = plsc.get_sparse_core_info()
NC, NS = _INFO.num_cores, _INFO.num_subcores
NW = NC * NS

B_TOTAL = BATCH * HIST
B_PER_W = B_TOTAL // NW
CHUNK = 1024
NCHUNKS = B_PER_W // CHUNK

assert B_PER_W * NW == B_TOTAL
assert NCHUNKS * CHUNK == B_PER_W


@functools.partial(
    pl.kernel,
    mesh=plsc.VectorSubcoreMesh(core_axis_name="c", subcore_axis_name="s"),
    out_type=jax.ShapeDtypeStruct((B_TOTAL, EMBED_DIM), jnp.float32),
    scratch_types=[
        pltpu.VMEM((CHUNK,), jnp.int32),
        pltpu.VMEM((CHUNK, EMBED_DIM), jnp.float32),
        pltpu.SemaphoreType.DMA,
    ],
    compiler_params=pltpu.CompilerParams(use_tc_tiling_on_sc=False),
)
def _gather_kernel(idx_hbm, table_hbm, out_hbm, idx_v, rows_v, sem):
    wid = lax.axis_index("s") * NC + lax.axis_index("c")
    wbase = wid * B_PER_W

    def body(i, carry):
        base = pl.multiple_of(wbase + i * CHUNK, 8)
        pltpu.sync_copy(idx_hbm.at[pl.ds(base, CHUNK)], idx_v)
        pltpu.async_copy(table_hbm.at[idx_v], rows_v, sem).wait()
        pltpu.sync_copy(rows_v, out_hbm.at[pl.ds(base, CHUNK)])
        return carry

    lax.fori_loop(0, NCHUNKS, body, 0)


def kernel(input_ids, table):
    flat = input_ids.reshape(-1)
    out = _gather_kernel(flat, table)
    return out.reshape(BATCH, HIST, EMBED_DIM)


# speedup vs baseline: 1.0161x; 1.0161x over previous
import functools

import jax
import jax.numpy as jnp
from jax import lax
from jax.experimental import pallas as pl
from jax.experimental.pallas import tpu as pltpu
from jax.experimental.pallas import tpu_sc as plsc

VOCAB = 1000000
EMBED_DIM = 64
BATCH = 16384
HIST = 200

_INFO = plsc.get_sparse_core_info()
NC, NS = _INFO.num_cores, _INFO.num_subcores
NW = NC * NS

B_TOTAL = BATCH * HIST
B_PER_W = B_TOTAL // NW
CHUNK = 800
NCHUNKS = B_PER_W // CHUNK
NPAIR = NCHUNKS // 2

assert B_PER_W * NW == B_TOTAL
assert NCHUNKS * CHUNK == B_PER_W
assert NCHUNKS % 2 == 0
assert 2 * (CHUNK + CHUNK * EMBED_DIM) <= 131071


@functools.partial(
    pl.kernel,
    mesh=plsc.VectorSubcoreMesh(core_axis_name="c", subcore_axis_name="s"),
    out_type=jax.ShapeDtypeStruct((B_TOTAL, EMBED_DIM), jnp.float32),
    scratch_types=[
        pltpu.VMEM((CHUNK,), jnp.int32),
        pltpu.VMEM((CHUNK,), jnp.int32),
        pltpu.VMEM((CHUNK, EMBED_DIM), jnp.float32),
        pltpu.VMEM((CHUNK, EMBED_DIM), jnp.float32),
        pltpu.SemaphoreType.DMA,
        pltpu.SemaphoreType.DMA,
        pltpu.SemaphoreType.DMA,
        pltpu.SemaphoreType.DMA,
    ],
    compiler_params=pltpu.CompilerParams(use_tc_tiling_on_sc=False),
)
def _gather_kernel(idx_hbm, table_hbm, out_hbm,
                   idx0, idx1, rows0, rows1, sg0, sg1, ss0, ss1):
    wid = lax.axis_index("s") * NC + lax.axis_index("c")
    wbase = wid * B_PER_W

    def cbase(k):
        return pl.multiple_of(wbase + k * CHUNK, 8)

    idx_v = (idx0, idx1)
    rows_v = (rows0, rows1)
    sg = (sg0, sg1)
    ss = (ss0, ss1)

    def half_step(j, s, guard_prev):
        k = 2 * j + s
        o = 1 - s

        def wait_reuse():
            pltpu.make_async_copy(
                rows_v[s], out_hbm.at[pl.ds(cbase(k - 2), CHUNK)], ss[s]
            ).wait()

        def finish_prev():
            pltpu.make_async_copy(
                table_hbm.at[idx_v[o]], rows_v[o], sg[o]
            ).wait()
            pltpu.async_copy(
                rows_v[o], out_hbm.at[pl.ds(cbase(k - 1), CHUNK)], ss[o]
            )

        if guard_prev:
            pl.when(j >= 1)(wait_reuse)
        else:
            wait_reuse()
        pltpu.sync_copy(idx_hbm.at[pl.ds(cbase(k), CHUNK)], idx_v[s])
        pltpu.async_copy(table_hbm.at[idx_v[s]], rows_v[s], sg[s])
        if guard_prev and s == 0:
            pl.when(j >= 1)(finish_prev)
        else:
            finish_prev()

    def body(j, carry):
        half_step(j, 0, guard_prev=True)
        half_step(j, 1, guard_prev=True)
        return carry

    lax.fori_loop(0, NPAIR, body, 0)

    last = NCHUNKS - 1
    pltpu.make_async_copy(table_hbm.at[idx1], rows1, sg1).wait()
    pltpu.async_copy(rows1, out_hbm.at[pl.ds(cbase(last), CHUNK)], ss1)
    pltpu.make_async_copy(
        rows0, out_hbm.at[pl.ds(cbase(last - 1), CHUNK)], ss0).wait()
    pltpu.make_async_copy(
        rows1, out_hbm.at[pl.ds(cbase(last), CHUNK)], ss1).wait()


def kernel(input_ids, table):
    flat = input_ids.reshape(-1)
    out = _gather_kernel(flat, table)
    return out.reshape(BATCH, HIST, EMBED_DIM)
